# baseline (speedup 1.0000x reference)
import functools

import jax
import jax.numpy as jnp
from jax import lax
from jax.experimental import pallas as pl
from jax.experimental.pallas import tpu as pltpu
from jax.experimental.pallas import tpu_sc as plsc

_NUM_BINS = 16
_BOUND = 5.0
_N = 16777216
_NW = 32
_PER_W = _N // _NW
_CHUNK = 16384
_NCHUNK = _PER_W // _CHUNK
_NCHUNK2 = _NCHUNK // 2
_NVEC = _CHUNK // 16
_LUT_SIZE = 2048
_HALF_DOM = 5.5
_SCALE = (_LUT_SIZE - 1) / (2.0 * _HALF_DOM)
_TROW = 32


def _build_tables(unnorm_widths, unnorm_heights, unnorm_derivatives):
    widths = jax.nn.softmax(unnorm_widths, axis=-1)
    widths = 0.001 + (1.0 - 0.001 * _NUM_BINS) * widths
    widths = widths * 2.0 * _BOUND
    heights = jax.nn.softmax(unnorm_heights, axis=-1)
    heights = 0.001 + (1.0 - 0.001 * _NUM_BINS) * heights
    heights = heights * 2.0 * _BOUND
    derivs = jax.nn.softplus(unnorm_derivatives) + 0.001
    neg_b = jnp.array([-_BOUND], dtype=jnp.float32)
    knot_x = jnp.concatenate([neg_b, -_BOUND + jnp.cumsum(widths)])
    knot_y = jnp.concatenate([neg_b, -_BOUND + jnp.cumsum(heights)])

    w = widths
    h = heights
    d0 = derivs[:-1]
    d1 = derivs[1:]
    s = h / w
    g = d0 + d1 - 2.0 * s
    alpha = h * (s - d0) / (w * w * s)
    beta = h * d0 / (w * s)
    c1 = g / (w * s)
    c2 = -g / (w * w * s)

    zeros1 = jnp.zeros((1,), jnp.float32)
    inf1 = jnp.array([3e38], jnp.float32)
    xk_t = jnp.concatenate([neg_b, knot_x[:-1], knot_x[-1:]])
    yk_t = jnp.concatenate([neg_b, knot_y[:-1], knot_y[-1:]])
    al_t = jnp.concatenate([zeros1, alpha, zeros1])
    bt_t = jnp.concatenate([derivs[:1], beta, derivs[-1:]])
    c1_t = jnp.concatenate([zeros1, c1, zeros1])
    c2_t = jnp.concatenate([zeros1, c2, zeros1])

    bnd = jnp.array([_BOUND], dtype=jnp.float32)
    tau = jnp.concatenate([neg_b, knot_x[1:_NUM_BINS], bnd])
    tn_t = jnp.concatenate([tau, inf1])

    edges = jnp.arange(_LUT_SIZE, dtype=jnp.float32) / jnp.float32(_SCALE) \
        - jnp.float32(_HALF_DOM)
    lut = jnp.sum(tau[None, :] < edges[:, None], axis=1).astype(jnp.int32)

    tn_cell = tn_t[lut]
    be = jnp.stack([lut, jnp.minimum(lut + 1, 17)], axis=1).reshape(-1)

    def cell(t):
        return t[be]

    return (tn_cell, cell(xk_t), cell(yk_t), cell(al_t), cell(bt_t),
            cell(c1_t), cell(c2_t))


def _spline_body(x_hbm, tn_hbm, xk_hbm, yk_hbm, al_hbm, bt_hbm, c1_hbm,
                 c2_hbm, out_hbm, tn_v, xk_v, yk_v, al_v, bt_v, c1_v, c2_v,
                 xb0, xb1, yb0, yb1, si0, si1, so0, so1):
    wid = lax.axis_index("s") * 2 + lax.axis_index("c")
    base = wid * _PER_W
    pltpu.sync_copy(tn_hbm, tn_v)
    pltpu.sync_copy(xk_hbm, xk_v)
    pltpu.sync_copy(yk_hbm, yk_v)
    pltpu.sync_copy(al_hbm, al_v)
    pltpu.sync_copy(bt_hbm, bt_v)
    pltpu.sync_copy(c1_hbm, c1_v)
    pltpu.sync_copy(c2_hbm, c2_v)

    xbufs = (xb0, xb1)
    ybufs = (yb0, yb1)
    sins = (si0, si1)
    souts = (so0, so1)

    half = jnp.float32(_HALF_DOM)
    scale = jnp.float32(_SCALE)
    one = jnp.float32(1.0)

    def xsl(ci):
        return x_hbm.at[pl.ds(base + ci * _CHUNK, _CHUNK)]

    def osl(ci):
        return out_hbm.at[pl.ds(base + ci * _CHUNK, _CHUNK)]

    def compute(xbuf, ybuf):
        @plsc.parallel_loop(0, _NVEC, unroll=8)
        def _(i):
            xv = xbuf[pl.ds(i * 16, 16)]
            xj = jnp.minimum(jnp.maximum(xv, -half), half)
            j = ((xj + half) * scale).astype(jnp.int32)
            tn = plsc.load_gather(tn_v, [j])
            j2 = j + j + jnp.where(tn < xv, 1, 0)
            xk = plsc.load_gather(xk_v, [j2])
            yk = plsc.load_gather(yk_v, [j2])
            al = plsc.load_gather(al_v, [j2])
            bt = plsc.load_gather(bt_v, [j2])
            k1 = plsc.load_gather(c1_v, [j2])
            k2 = plsc.load_gather(c2_v, [j2])
            u = xv - xk
            num = (al * u + bt) * u
            den = (k2 * u + k1) * u + one
            ybuf[pl.ds(i * 16, 16)] = yk + num / den

    pltpu.async_copy(xsl(0), xb0, si0)
    pltpu.async_copy(xsl(1), xb1, si1)

    def chunk2(c2, carry):
        for b in range(2):
            ci = c2 * 2 + b
            pltpu.make_async_copy(xsl(ci), xbufs[b], sins[b]).wait()

            @pl.when(c2 >= 1)
            def _():
                pltpu.make_async_copy(ybufs[b], osl(ci - 2), souts[b]).wait()

            compute(xbufs[b], ybufs[b])
            pltpu.async_copy(ybufs[b], osl(ci), souts[b])

            @pl.when(c2 + 1 < _NCHUNK2)
            def _():
                pltpu.async_copy(xsl(ci + 2), xbufs[b], sins[b])
        return carry

    lax.fori_loop(0, _NCHUNK2, chunk2, 0)
    for b in range(2):
        ci = (_NCHUNK2 - 1) * 2 + b
        pltpu.make_async_copy(ybufs[b], osl(ci), souts[b]).wait()


_mesh = plsc.VectorSubcoreMesh(
    core_axis_name="c", subcore_axis_name="s", num_cores=2, num_subcores=16)

_spline_call = functools.partial(
    pl.kernel,
    out_type=jax.ShapeDtypeStruct((_N,), jnp.float32),
    mesh=_mesh,
    scratch_types=[
        pltpu.VMEM((_LUT_SIZE,), jnp.float32),
        pltpu.VMEM((2 * _LUT_SIZE,), jnp.float32),
        pltpu.VMEM((2 * _LUT_SIZE,), jnp.float32),
        pltpu.VMEM((2 * _LUT_SIZE,), jnp.float32),
        pltpu.VMEM((2 * _LUT_SIZE,), jnp.float32),
        pltpu.VMEM((2 * _LUT_SIZE,), jnp.float32),
        pltpu.VMEM((2 * _LUT_SIZE,), jnp.float32),
        pltpu.VMEM((_CHUNK,), jnp.float32),
        pltpu.VMEM((_CHUNK,), jnp.float32),
        pltpu.VMEM((_CHUNK,), jnp.float32),
        pltpu.VMEM((_CHUNK,), jnp.float32),
        pltpu.SemaphoreType.DMA,
        pltpu.SemaphoreType.DMA,
        pltpu.SemaphoreType.DMA,
        pltpu.SemaphoreType.DMA,
    ],
    compiler_params=pltpu.CompilerParams(needs_layout_passes=False),
)(_spline_body)


def kernel(x, unnorm_widths, unnorm_heights, unnorm_derivatives):
    tabs = _build_tables(unnorm_widths, unnorm_heights, unnorm_derivatives)
    return _spline_call(x, *tabs)


# speedup vs baseline: 1.5789x; 1.5789x over previous
import functools

import jax
import jax.numpy as jnp
from jax import lax
from jax.experimental import pallas as pl
from jax.experimental.pallas import tpu as pltpu
from jax.experimental.pallas import tpu_sc as plsc

_NUM_BINS = 16
_BOUND = 5.0
_N = 16777216
_NW = 32
_PER_W = _N // _NW
_CHUNK = 16384
_NCHUNK = _PER_W // _CHUNK
_NCHUNK2 = _NCHUNK // 2
_NVEC = _CHUNK // 16
_LUT_SIZE = 2048
_HALF_DOM = 5.5
_SCALE = (_LUT_SIZE - 1) / (2.0 * _HALF_DOM)
_TROW = 32


def _build_tables(unnorm_widths, unnorm_heights, unnorm_derivatives):
    widths = jax.nn.softmax(unnorm_widths, axis=-1)
    widths = 0.001 + (1.0 - 0.001 * _NUM_BINS) * widths
    widths = widths * 2.0 * _BOUND
    heights = jax.nn.softmax(unnorm_heights, axis=-1)
    heights = 0.001 + (1.0 - 0.001 * _NUM_BINS) * heights
    heights = heights * 2.0 * _BOUND
    derivs = jax.nn.softplus(unnorm_derivatives) + 0.001
    neg_b = jnp.array([-_BOUND], dtype=jnp.float32)
    knot_x = jnp.concatenate([neg_b, -_BOUND + jnp.cumsum(widths)])
    knot_y = jnp.concatenate([neg_b, -_BOUND + jnp.cumsum(heights)])

    w = widths
    h = heights
    d0 = derivs[:-1]
    d1 = derivs[1:]
    s = h / w
    g = d0 + d1 - 2.0 * s
    alpha = h * (s - d0) / (w * w * s)
    beta = h * d0 / (w * s)
    c1 = g / (w * s)
    c2 = -g / (w * w * s)

    zeros1 = jnp.zeros((1,), jnp.float32)
    inf1 = jnp.array([3e38], jnp.float32)
    xk_t = jnp.concatenate([neg_b, knot_x[:-1], knot_x[-1:]])
    yk_t = jnp.concatenate([neg_b, knot_y[:-1], knot_y[-1:]])
    al_t = jnp.concatenate([zeros1, alpha, zeros1])
    bt_t = jnp.concatenate([derivs[:1], beta, derivs[-1:]])
    c1_t = jnp.concatenate([zeros1, c1, zeros1])
    c2_t = jnp.concatenate([zeros1, c2, zeros1])

    bnd = jnp.array([_BOUND], dtype=jnp.float32)
    tau = jnp.concatenate([neg_b, knot_x[1:_NUM_BINS], bnd])
    tn_t = jnp.concatenate([tau, inf1])

    edges = jnp.arange(_LUT_SIZE, dtype=jnp.float32) / jnp.float32(_SCALE) \
        - jnp.float32(_HALF_DOM)
    lut = jnp.sum(tau[None, :] < edges[:, None], axis=1).astype(jnp.int32)

    def pad(t):
        return jnp.pad(t, (0, _TROW - t.shape[0]))

    return (lut, pad(tn_t), pad(xk_t), pad(yk_t), pad(al_t), pad(bt_t),
            pad(c1_t), pad(c2_t))


def _spline_body(x_hbm, lut_hbm, tn_hbm, xk_hbm, yk_hbm, al_hbm, bt_hbm,
                 c1_hbm, c2_hbm, out_hbm, lut_v, tn_v, xk_v, yk_v, al_v,
                 bt_v, c1_v, c2_v, xb0, xb1, yb0, yb1, si0, si1, so0, so1):
    wid = lax.axis_index("s") * 2 + lax.axis_index("c")
    base = wid * _PER_W
    pltpu.sync_copy(lut_hbm, lut_v)
    pltpu.sync_copy(tn_hbm, tn_v)
    pltpu.sync_copy(xk_hbm, xk_v)
    pltpu.sync_copy(yk_hbm, yk_v)
    pltpu.sync_copy(al_hbm, al_v)
    pltpu.sync_copy(bt_hbm, bt_v)
    pltpu.sync_copy(c1_hbm, c1_v)
    pltpu.sync_copy(c2_hbm, c2_v)

    xbufs = (xb0, xb1)
    ybufs = (yb0, yb1)
    sins = (si0, si1)
    souts = (so0, so1)

    half = jnp.float32(_HALF_DOM)
    scale = jnp.float32(_SCALE)
    one = jnp.float32(1.0)

    def xsl(ci):
        return x_hbm.at[pl.ds(base + ci * _CHUNK, _CHUNK)]

    def osl(ci):
        return out_hbm.at[pl.ds(base + ci * _CHUNK, _CHUNK)]

    def compute(xbuf, ybuf):
        @plsc.parallel_loop(0, _NVEC, unroll=8)
        def _(i):
            xv = xbuf[pl.ds(i * 16, 16)]
            xj = jnp.minimum(jnp.maximum(xv, -half), half)
            j = ((xj + half) * scale).astype(jnp.int32)
            c = plsc.load_gather(lut_v, [j])
            tn = plsc.load_gather(tn_v, [c])
            be = c + jnp.where(tn < xv, 1, 0)
            xk = plsc.load_gather(xk_v, [be])
            yk = plsc.load_gather(yk_v, [be])
            al = plsc.load_gather(al_v, [be])
            bt = plsc.load_gather(bt_v, [be])
            k1 = plsc.load_gather(c1_v, [be])
            k2 = plsc.load_gather(c2_v, [be])
            u = xv - xk
            num = (al * u + bt) * u
            den = (k2 * u + k1) * u + one
            ybuf[pl.ds(i * 16, 16)] = yk + num / den

    pltpu.async_copy(xsl(0), xb0, si0)
    pltpu.async_copy(xsl(1), xb1, si1)

    def chunk2(c2, carry):
        for b in range(2):
            ci = c2 * 2 + b
            pltpu.make_async_copy(xsl(ci), xbufs[b], sins[b]).wait()

            @pl.when(c2 >= 1)
            def _():
                pltpu.make_async_copy(ybufs[b], osl(ci - 2), souts[b]).wait()

            compute(xbufs[b], ybufs[b])
            pltpu.async_copy(ybufs[b], osl(ci), souts[b])

            @pl.when(c2 + 1 < _NCHUNK2)
            def _():
                pltpu.async_copy(xsl(ci + 2), xbufs[b], sins[b])
        return carry

    lax.fori_loop(0, _NCHUNK2, chunk2, 0)
    for b in range(2):
        ci = (_NCHUNK2 - 1) * 2 + b
        pltpu.make_async_copy(ybufs[b], osl(ci), souts[b]).wait()


_mesh = plsc.VectorSubcoreMesh(
    core_axis_name="c", subcore_axis_name="s", num_cores=2, num_subcores=16)

_spline_call = functools.partial(
    pl.kernel,
    out_type=jax.ShapeDtypeStruct((_N,), jnp.float32),
    mesh=_mesh,
    scratch_types=[
        pltpu.VMEM((_LUT_SIZE,), jnp.int32),
        pltpu.VMEM((_TROW,), jnp.float32),
        pltpu.VMEM((_TROW,), jnp.float32),
        pltpu.VMEM((_TROW,), jnp.float32),
        pltpu.VMEM((_TROW,), jnp.float32),
        pltpu.VMEM((_TROW,), jnp.float32),
        pltpu.VMEM((_TROW,), jnp.float32),
        pltpu.VMEM((_TROW,), jnp.float32),
        pltpu.VMEM((_CHUNK,), jnp.float32),
        pltpu.VMEM((_CHUNK,), jnp.float32),
        pltpu.VMEM((_CHUNK,), jnp.float32),
        pltpu.VMEM((_CHUNK,), jnp.float32),
        pltpu.SemaphoreType.DMA,
        pltpu.SemaphoreType.DMA,
        pltpu.SemaphoreType.DMA,
        pltpu.SemaphoreType.DMA,
    ],
    compiler_params=pltpu.CompilerParams(needs_layout_passes=False),
)(_spline_body)


def kernel(x, unnorm_widths, unnorm_heights, unnorm_derivatives):
    tabs = _build_tables(unnorm_widths, unnorm_heights, unnorm_derivatives)
    return _spline_call(x, *tabs)


# speedup vs baseline: 1.6623x; 1.0528x over previous
import functools

import jax
import jax.numpy as jnp
from jax import lax
from jax.experimental import pallas as pl
from jax.experimental.pallas import tpu as pltpu
from jax.experimental.pallas import tpu_sc as plsc

_NUM_BINS = 16
_BOUND = 5.0
_N = 16777216
_NW = 32
_PER_W = _N // _NW
_CHUNK = 16384
_NCHUNK = _PER_W // _CHUNK
_NCHUNK2 = _NCHUNK // 2
_NVEC = _CHUNK // 16
_LUT_SIZE = 2048
_HALF_DOM = 5.5
_SCALE = (_LUT_SIZE - 1) / (2.0 * _HALF_DOM)
_TROW = 32


def _build_tables(unnorm_widths, unnorm_heights, unnorm_derivatives):
    widths = jax.nn.softmax(unnorm_widths, axis=-1)
    widths = 0.001 + (1.0 - 0.001 * _NUM_BINS) * widths
    widths = widths * 2.0 * _BOUND
    heights = jax.nn.softmax(unnorm_heights, axis=-1)
    heights = 0.001 + (1.0 - 0.001 * _NUM_BINS) * heights
    heights = heights * 2.0 * _BOUND
    derivs = jax.nn.softplus(unnorm_derivatives) + 0.001
    neg_b = jnp.array([-_BOUND], dtype=jnp.float32)
    knot_x = jnp.concatenate([neg_b, -_BOUND + jnp.cumsum(widths)])
    knot_y = jnp.concatenate([neg_b, -_BOUND + jnp.cumsum(heights)])

    w = widths
    h = heights
    d0 = derivs[:-1]
    d1 = derivs[1:]
    s = h / w
    g = d0 + d1 - 2.0 * s
    alpha = h * (s - d0) / (w * w * s)
    beta = h * d0 / (w * s)
    c1 = g / (w * s)
    c2 = -g / (w * w * s)

    zeros1 = jnp.zeros((1,), jnp.float32)
    inf1 = jnp.array([3e38], jnp.float32)
    xk_t = jnp.concatenate([neg_b, knot_x[:-1], knot_x[-1:]])
    yk_t = jnp.concatenate([neg_b, knot_y[:-1], knot_y[-1:]])
    al_t = jnp.concatenate([zeros1, alpha, zeros1])
    bt_t = jnp.concatenate([derivs[:1], beta, derivs[-1:]])
    c1_t = jnp.concatenate([zeros1, c1, zeros1])
    c2_t = jnp.concatenate([zeros1, c2, zeros1])

    bnd = jnp.array([_BOUND], dtype=jnp.float32)
    tau = jnp.concatenate([neg_b, knot_x[1:_NUM_BINS], bnd])
    tn_t = jnp.concatenate([tau, inf1])

    edges = jnp.arange(_LUT_SIZE, dtype=jnp.float32) / jnp.float32(_SCALE) \
        - jnp.float32(_HALF_DOM)
    lut = jnp.sum(tau[None, :] < edges[:, None], axis=1).astype(jnp.int32)

    tn_bits = jax.lax.bitcast_convert_type(tn_t[lut], jnp.int32)
    lut_packed = (tn_bits & jnp.int32(-32)) | lut

    def pad(t):
        return jnp.pad(t, (0, _TROW - t.shape[0]))

    return (lut_packed, pad(xk_t), pad(yk_t), pad(al_t), pad(bt_t),
            pad(c1_t), pad(c2_t))


def _spline_body(x_hbm, lut_hbm, xk_hbm, yk_hbm, al_hbm, bt_hbm,
                 c1_hbm, c2_hbm, out_hbm, lut_v, xk_v, yk_v, al_v,
                 bt_v, c1_v, c2_v, xb0, xb1, yb0, yb1, si0, si1, so0, so1):
    wid = lax.axis_index("s") * 2 + lax.axis_index("c")
    base = wid * _PER_W
    pltpu.sync_copy(lut_hbm, lut_v)
    pltpu.sync_copy(xk_hbm, xk_v)
    pltpu.sync_copy(yk_hbm, yk_v)
    pltpu.sync_copy(al_hbm, al_v)
    pltpu.sync_copy(bt_hbm, bt_v)
    pltpu.sync_copy(c1_hbm, c1_v)
    pltpu.sync_copy(c2_hbm, c2_v)

    xbufs = (xb0, xb1)
    ybufs = (yb0, yb1)
    sins = (si0, si1)
    souts = (so0, so1)

    half = jnp.float32(_HALF_DOM)
    scale = jnp.float32(_SCALE)
    one = jnp.float32(1.0)

    def xsl(ci):
        return x_hbm.at[pl.ds(base + ci * _CHUNK, _CHUNK)]

    def osl(ci):
        return out_hbm.at[pl.ds(base + ci * _CHUNK, _CHUNK)]

    def compute(xbuf, ybuf):
        @plsc.parallel_loop(0, _NVEC, unroll=8)
        def _(i):
            xv = xbuf[pl.ds(i * 16, 16)]
            xj = jnp.minimum(jnp.maximum(xv, -half), half)
            j = ((xj + half) * scale).astype(jnp.int32)
            e = plsc.load_gather(lut_v, [j])
            c = e & jnp.int32(31)
            tn = plsc.bitcast(e & jnp.int32(-32), jnp.float32)
            be = c + jnp.where(tn < xv, 1, 0)
            xk = plsc.load_gather(xk_v, [be])
            yk = plsc.load_gather(yk_v, [be])
            al = plsc.load_gather(al_v, [be])
            bt = plsc.load_gather(bt_v, [be])
            k1 = plsc.load_gather(c1_v, [be])
            k2 = plsc.load_gather(c2_v, [be])
            u = xv - xk
            num = (al * u + bt) * u
            den = (k2 * u + k1) * u + one
            ybuf[pl.ds(i * 16, 16)] = yk + num / den

    pltpu.async_copy(xsl(0), xb0, si0)
    pltpu.async_copy(xsl(1), xb1, si1)

    def chunk2(c2, carry):
        for b in range(2):
            ci = c2 * 2 + b
            pltpu.make_async_copy(xsl(ci), xbufs[b], sins[b]).wait()

            @pl.when(c2 >= 1)
            def _():
                pltpu.make_async_copy(ybufs[b], osl(ci - 2), souts[b]).wait()

            compute(xbufs[b], ybufs[b])
            pltpu.async_copy(ybufs[b], osl(ci), souts[b])

            @pl.when(c2 + 1 < _NCHUNK2)
            def _():
                pltpu.async_copy(xsl(ci + 2), xbufs[b], sins[b])
        return carry

    lax.fori_loop(0, _NCHUNK2, chunk2, 0)
    for b in range(2):
        ci = (_NCHUNK2 - 1) * 2 + b
        pltpu.make_async_copy(ybufs[b], osl(ci), souts[b]).wait()


_mesh = plsc.VectorSubcoreMesh(
    core_axis_name="c", subcore_axis_name="s", num_cores=2, num_subcores=16)

_spline_call = functools.partial(
    pl.kernel,
    out_type=jax.ShapeDtypeStruct((_N,), jnp.float32),
    mesh=_mesh,
    scratch_types=[
        pltpu.VMEM((_LUT_SIZE,), jnp.int32),
        pltpu.VMEM((_TROW,), jnp.float32),
        pltpu.VMEM((_TROW,), jnp.float32),
        pltpu.VMEM((_TROW,), jnp.float32),
        pltpu.VMEM((_TROW,), jnp.float32),
        pltpu.VMEM((_TROW,), jnp.float32),
        pltpu.VMEM((_TROW,), jnp.float32),
        pltpu.VMEM((_CHUNK,), jnp.float32),
        pltpu.VMEM((_CHUNK,), jnp.float32),
        pltpu.VMEM((_CHUNK,), jnp.float32),
        pltpu.VMEM((_CHUNK,), jnp.float32),
        pltpu.SemaphoreType.DMA,
        pltpu.SemaphoreType.DMA,
        pltpu.SemaphoreType.DMA,
        pltpu.SemaphoreType.DMA,
    ],
    compiler_params=pltpu.CompilerParams(needs_layout_passes=False),
)(_spline_body)


def kernel(x, unnorm_widths, unnorm_heights, unnorm_derivatives):
    tabs = _build_tables(unnorm_widths, unnorm_heights, unnorm_derivatives)
    return _spline_call(x, *tabs)


# speedup vs baseline: 1.8233x; 1.0968x over previous
import functools

import jax
import jax.numpy as jnp
from jax import lax
from jax.experimental import pallas as pl
from jax.experimental.pallas import tpu as pltpu
from jax.experimental.pallas import tpu_sc as plsc

_NUM_BINS = 16
_BOUND = 5.0
_N = 16777216
_NW = 32
_PER_W = _N // _NW
_CHUNK = 16384
_NCHUNK = _PER_W // _CHUNK
_NCHUNK2 = _NCHUNK // 2
_NVEC = _CHUNK // 16
_LUT_SIZE = 2048
_HALF_DOM = 5.5
_SCALE = (_LUT_SIZE - 1) / (2.0 * _HALF_DOM)
_TROW = 32


def _build_tables(unnorm_widths, unnorm_heights, unnorm_derivatives):
    widths = jax.nn.softmax(unnorm_widths, axis=-1)
    widths = 0.001 + (1.0 - 0.001 * _NUM_BINS) * widths
    widths = widths * 2.0 * _BOUND
    heights = jax.nn.softmax(unnorm_heights, axis=-1)
    heights = 0.001 + (1.0 - 0.001 * _NUM_BINS) * heights
    heights = heights * 2.0 * _BOUND
    derivs = jax.nn.softplus(unnorm_derivatives) + 0.001
    neg_b = jnp.array([-_BOUND], dtype=jnp.float32)
    knot_x = jnp.concatenate([neg_b, -_BOUND + jnp.cumsum(widths)])
    knot_y = jnp.concatenate([neg_b, -_BOUND + jnp.cumsum(heights)])

    w = widths
    h = heights
    d0 = derivs[:-1]
    d1 = derivs[1:]
    s = h / w
    g = d0 + d1 - 2.0 * s
    alpha = h * (s - d0) / (w * w * s)
    beta = h * d0 / (w * s)
    c1 = g / (w * s)
    c2 = -g / (w * w * s)

    zeros1 = jnp.zeros((1,), jnp.float32)
    inf1 = jnp.array([3e38], jnp.float32)
    xk_t = jnp.concatenate([neg_b, knot_x[:-1], knot_x[-1:]])
    yk_t = jnp.concatenate([neg_b, knot_y[:-1], knot_y[-1:]])
    al_t = jnp.concatenate([zeros1, alpha, zeros1])
    bt_t = jnp.concatenate([derivs[:1], beta, derivs[-1:]])
    c1_t = jnp.concatenate([zeros1, c1, zeros1])
    c2_t = jnp.concatenate([zeros1, c2, zeros1])

    bnd = jnp.array([_BOUND], dtype=jnp.float32)
    tau = jnp.concatenate([neg_b, knot_x[1:_NUM_BINS], bnd])
    tn_t = jnp.concatenate([tau, inf1])

    edges = (jnp.arange(_LUT_SIZE, dtype=jnp.float32) - 0.5) \
        / jnp.float32(_SCALE) - jnp.float32(_HALF_DOM)
    lut = jnp.sum(tau[None, :] < edges[:, None], axis=1).astype(jnp.int32)

    tn_bits = jax.lax.bitcast_convert_type(tn_t[lut], jnp.int32)
    lut_packed = (tn_bits & jnp.int32(-32)) | lut

    k1_bits = jax.lax.bitcast_convert_type(c1_t.astype(jnp.bfloat16), jnp.int16)
    k2_bits = jax.lax.bitcast_convert_type(c2_t.astype(jnp.bfloat16), jnp.int16)
    kk_t = (k1_bits.astype(jnp.int32) << 16) | (
        k2_bits.astype(jnp.int32) & jnp.int32(0xFFFF))

    def pad(t):
        return jnp.pad(t, (0, _TROW - t.shape[0]))

    return (lut_packed, pad(xk_t), pad(yk_t), pad(al_t), pad(bt_t),
            pad(kk_t))


def _spline_body(x_hbm, lut_hbm, xk_hbm, yk_hbm, al_hbm, bt_hbm,
                 kk_hbm, out_hbm, lut_v, xk_v, yk_v, al_v,
                 bt_v, kk_v, xb0, xb1, yb0, yb1, si0, si1, so0, so1):
    wid = lax.axis_index("s") * 2 + lax.axis_index("c")
    base = wid * _PER_W
    pltpu.sync_copy(lut_hbm, lut_v)
    pltpu.sync_copy(xk_hbm, xk_v)
    pltpu.sync_copy(yk_hbm, yk_v)
    pltpu.sync_copy(al_hbm, al_v)
    pltpu.sync_copy(bt_hbm, bt_v)
    pltpu.sync_copy(kk_hbm, kk_v)

    xbufs = (xb0, xb1)
    ybufs = (yb0, yb1)
    sins = (si0, si1)
    souts = (so0, so1)

    half = jnp.float32(_HALF_DOM)
    scale = jnp.float32(_SCALE)
    one = jnp.float32(1.0)
    magic = jnp.float32(2.0 ** 23 + _HALF_DOM * _SCALE)

    def xsl(ci):
        return x_hbm.at[pl.ds(base + ci * _CHUNK, _CHUNK)]

    def osl(ci):
        return out_hbm.at[pl.ds(base + ci * _CHUNK, _CHUNK)]

    def compute(xbuf, ybuf):
        @plsc.parallel_loop(0, _NVEC, unroll=8)
        def _(i):
            xv = xbuf[pl.ds(i * 16, 16)]
            xj = jnp.minimum(jnp.maximum(xv, -half), half)
            j = plsc.bitcast(xj * scale + magic, jnp.int32) & jnp.int32(
                _LUT_SIZE - 1)
            e = plsc.load_gather(lut_v, [j])
            c = e & jnp.int32(31)
            tn = plsc.bitcast(e, jnp.float32)
            be = jnp.where(tn < xv, c + 1, c)
            xk = plsc.load_gather(xk_v, [be])
            yk = plsc.load_gather(yk_v, [be])
            al = plsc.load_gather(al_v, [be])
            bt = plsc.load_gather(bt_v, [be])
            kk = plsc.load_gather(kk_v, [be])
            k1 = plsc.bitcast(kk & jnp.int32(-65536), jnp.float32)
            k2 = plsc.bitcast(kk << 16, jnp.float32)
            u = xv - xk
            num = (al * u + bt) * u
            den = (k2 * u + k1) * u + one
            ybuf[pl.ds(i * 16, 16)] = yk + num / den

    pltpu.async_copy(xsl(0), xb0, si0)
    pltpu.async_copy(xsl(1), xb1, si1)

    def chunk2(c2, carry):
        for b in range(2):
            ci = c2 * 2 + b
            pltpu.make_async_copy(xsl(ci), xbufs[b], sins[b]).wait()

            @pl.when(c2 >= 1)
            def _():
                pltpu.make_async_copy(ybufs[b], osl(ci - 2), souts[b]).wait()

            compute(xbufs[b], ybufs[b])
            pltpu.async_copy(ybufs[b], osl(ci), souts[b])

            @pl.when(c2 + 1 < _NCHUNK2)
            def _():
                pltpu.async_copy(xsl(ci + 2), xbufs[b], sins[b])
        return carry

    lax.fori_loop(0, _NCHUNK2, chunk2, 0)
    for b in range(2):
        ci = (_NCHUNK2 - 1) * 2 + b
        pltpu.make_async_copy(ybufs[b], osl(ci), souts[b]).wait()


_mesh = plsc.VectorSubcoreMesh(
    core_axis_name="c", subcore_axis_name="s", num_cores=2, num_subcores=16)

_spline_call = functools.partial(
    pl.kernel,
    out_type=jax.ShapeDtypeStruct((_N,), jnp.float32),
    mesh=_mesh,
    scratch_types=[
        pltpu.VMEM((_LUT_SIZE,), jnp.int32),
        pltpu.VMEM((_TROW,), jnp.float32),
        pltpu.VMEM((_TROW,), jnp.float32),
        pltpu.VMEM((_TROW,), jnp.float32),
        pltpu.VMEM((_TROW,), jnp.float32),
        pltpu.VMEM((_TROW,), jnp.int32),
        pltpu.VMEM((_CHUNK,), jnp.float32),
        pltpu.VMEM((_CHUNK,), jnp.float32),
        pltpu.VMEM((_CHUNK,), jnp.float32),
        pltpu.VMEM((_CHUNK,), jnp.float32),
        pltpu.SemaphoreType.DMA,
        pltpu.SemaphoreType.DMA,
        pltpu.SemaphoreType.DMA,
        pltpu.SemaphoreType.DMA,
    ],
    compiler_params=pltpu.CompilerParams(needs_layout_passes=False),
)(_spline_body)


def kernel(x, unnorm_widths, unnorm_heights, unnorm_derivatives):
    tabs = _build_tables(unnorm_widths, unnorm_heights, unnorm_derivatives)
    return _spline_call(x, *tabs)
